# Initial kernel scaffold; baseline (speedup 1.0000x reference)
import functools

import jax
import jax.numpy as jnp
from jax import lax
from jax.experimental import pallas as pl
from jax.experimental.pallas import tpu as pltpu
from jax.experimental.pallas import tpu_sc as plsc

_GROUP = 128
_GROUPS_PER_CHUNK = 10


@functools.lru_cache(maxsize=None)
def _make_gather(n_rows: int, emb: int, vocab: int):
    info = plsc.get_sparse_core_info()
    nw = info.num_cores * info.num_subcores
    chunk_rows = _GROUP * _GROUPS_PER_CHUNK
    assert n_rows % (nw * chunk_rows) == 0
    chunks_per_w = n_rows // (nw * chunk_rows)
    groups_per_w = chunks_per_w * _GROUPS_PER_CHUNK

    mesh = plsc.VectorSubcoreMesh(core_axis_name="c", subcore_axis_name="s")

    @functools.partial(
        pl.kernel,
        mesh=mesh,
        out_type=jax.ShapeDtypeStruct((n_rows, emb), jnp.float32),
        scratch_types=[
            pltpu.VMEM((_GROUPS_PER_CHUNK, _GROUP), jnp.int32),
            pltpu.VMEM((chunk_rows, emb), jnp.float32),
            pltpu.SemaphoreType.DMA,
        ],
    )
    def gather_k(table_hbm, idx_hbm, out_hbm, idx_v, rows_v, sem):
        cid = lax.axis_index("c")
        sid = lax.axis_index("s")
        wid = sid * info.num_cores + cid
        g0 = wid * groups_per_w

        def chunk_body(c, carry):
            gbase = g0 + c * _GROUPS_PER_CHUNK
            pltpu.sync_copy(idx_hbm.at[pl.ds(gbase, _GROUPS_PER_CHUNK)], idx_v)
            handles = []
            for j in range(_GROUPS_PER_CHUNK):
                handles.append(
                    pltpu.async_copy(
                        table_hbm.at[idx_v.at[j]],
                        rows_v.at[pl.ds(j * _GROUP, _GROUP)],
                        sem,
                    )
                )
            for h in handles:
                h.wait()
            pltpu.sync_copy(rows_v, out_hbm.at[pl.ds(gbase * _GROUP, chunk_rows)])
            return carry

        lax.fori_loop(0, chunks_per_w, chunk_body, 0)

    return gather_k


def _matmul_kernel(a_ref, w_ref, b_ref, o_ref):
    acc = lax.dot_general(
        a_ref[...], w_ref[...],
        dimension_numbers=(((1,), (1,)), ((), ())),
        preferred_element_type=jnp.float32,
    )
    o_ref[...] = acc + b_ref[...]


def _tc_matmul(flat, W, b2):
    batch, k = flat.shape
    out_dim = W.shape[0]
    bm = 512
    return pl.pallas_call(
        _matmul_kernel,
        grid=(batch // bm,),
        in_specs=[
            pl.BlockSpec((bm, k), lambda i: (i, 0)),
            pl.BlockSpec((out_dim, k), lambda i: (0, 0)),
            pl.BlockSpec((1, out_dim), lambda i: (0, 0)),
        ],
        out_specs=pl.BlockSpec((bm, out_dim), lambda i: (i, 0)),
        out_shape=jax.ShapeDtypeStruct((batch, out_dim), jnp.float32),
    )(flat, W, b2)


def kernel(x, table, W, b):
    batch, enc = x.shape
    vocab, emb = table.shape
    n_rows = batch * enc
    idx = x.reshape(n_rows // _GROUP, _GROUP).astype(jnp.int32)
    gathered = _make_gather(n_rows, emb, vocab)(table, idx)
    flat = gathered.reshape(batch, enc * emb)
    return _tc_matmul(flat, W, b.reshape(1, W.shape[0]))


# speedup vs baseline: 3.4970x; 3.4970x over previous
import functools

import jax
import jax.numpy as jnp
from jax import lax
from jax.experimental import pallas as pl
from jax.experimental.pallas import tpu as pltpu
from jax.experimental.pallas import tpu_sc as plsc

_GROUP = 128
_GROUPS_PER_CHUNK = 10
_EMBP = 64
_PAD = 128


@functools.lru_cache(maxsize=None)
def _make_gather(n_rows: int, vocab: int):
    info = plsc.get_sparse_core_info()
    nw = info.num_cores * info.num_subcores
    chunk_rows = _GROUP * _GROUPS_PER_CHUNK
    assert n_rows % (nw * chunk_rows) == 0
    chunks_per_w = n_rows // (nw * chunk_rows)

    mesh = plsc.VectorSubcoreMesh(core_axis_name="c", subcore_axis_name="s")

    @functools.partial(
        pl.kernel,
        mesh=mesh,
        out_type=jax.ShapeDtypeStruct((n_rows, _PAD), jnp.float32),
        scratch_types=[
            pltpu.VMEM((_GROUPS_PER_CHUNK, _GROUP), jnp.int32),
            pltpu.VMEM((chunk_rows, _EMBP), jnp.float32),
            pltpu.SemaphoreType.DMA,
        ],
        compiler_params=pltpu.CompilerParams(use_tc_tiling_on_sc=False),
    )
    def gather_k(table_hbm, idx_hbm, out_hbm, idx_v, rows_v, sem):
        cid = lax.axis_index("c")
        sid = lax.axis_index("s")
        wid = sid * info.num_cores + cid
        c0 = wid * chunks_per_w

        def chunk_body(c, carry):
            pltpu.sync_copy(idx_hbm.at[c0 + c], idx_v)
            handles = []
            for j in range(_GROUPS_PER_CHUNK):
                handles.append(
                    pltpu.async_copy(
                        table_hbm.at[idx_v.at[j]],
                        rows_v.at[pl.ds(j * _GROUP, _GROUP)],
                        sem,
                    )
                )
            for h in handles:
                h.wait()
            pltpu.sync_copy(
                rows_v,
                out_hbm.at[pl.ds((c0 + c) * chunk_rows, chunk_rows), pl.ds(0, _EMBP)],
            )
            return carry

        lax.fori_loop(0, chunks_per_w, chunk_body, 0)

    return gather_k


def _matmul_kernel(a_ref, w_ref, b_ref, o_ref):
    a = a_ref[...]
    lane = lax.broadcasted_iota(jnp.int32, a.shape, 1) % _PAD
    a = jnp.where(lane < _EMBP, a, 0.0)
    acc = lax.dot_general(
        a, w_ref[...],
        dimension_numbers=(((1,), (1,)), ((), ())),
        preferred_element_type=jnp.float32,
    )
    o_ref[...] = acc + b_ref[...]


def _tc_matmul(flat, Wp, b2):
    batch, kp = flat.shape
    out_dim = Wp.shape[0]
    bm = 256
    return pl.pallas_call(
        _matmul_kernel,
        grid=(batch // bm,),
        in_specs=[
            pl.BlockSpec((bm, kp), lambda i: (i, 0)),
            pl.BlockSpec((out_dim, kp), lambda i: (0, 0)),
            pl.BlockSpec((1, out_dim), lambda i: (0, 0)),
        ],
        out_specs=pl.BlockSpec((bm, out_dim), lambda i: (i, 0)),
        out_shape=jax.ShapeDtypeStruct((batch, out_dim), jnp.float32),
    )(flat, Wp, b2)


def kernel(x, table, W, b):
    batch, enc = x.shape
    vocab, emb = table.shape
    out_dim = W.shape[0]
    n_rows = batch * enc
    chunk_rows = _GROUP * _GROUPS_PER_CHUNK
    idx = x.reshape(n_rows // chunk_rows, _GROUPS_PER_CHUNK, _GROUP).astype(jnp.int32)

    table_p = jnp.pad(table, ((0, 0), (0, _EMBP - emb)))
    gathered = _make_gather(n_rows, vocab)(table_p, idx)
    flat = gathered.reshape(batch, enc * _PAD)

    Wp = jnp.pad(
        W.reshape(out_dim, enc, emb), ((0, 0), (0, 0), (0, _PAD - emb))
    ).reshape(out_dim, enc * _PAD)

    return _tc_matmul(flat, Wp, b.reshape(1, out_dim))
